# Initial kernel scaffold; baseline (speedup 1.0000x reference)
import jax
import jax.numpy as jnp
from jax.experimental import pallas as pl


def kernel(x, edge_index, edge_type, W0, root0, b0, W1, root1, b1, Wout, bout):
    raise NotImplementedError("write your pallas kernel here")


# speedup vs baseline: 18.9276x; 18.9276x over previous
import functools

import jax
import jax.numpy as jnp
from jax import lax
from jax.experimental import pallas as pl
from jax.experimental.pallas import tpu as pltpu
from jax.experimental.pallas import tpu_sc as plsc

N = 10000
E = 320000
D = 128
R = 8

NC = 2
NS = 16
NW = NC * NS
L = 16

C = 128
EW = ((E + NW - 1) // NW + C - 1) // C * C
E_PAD = EW * NW
NCHUNK = EW // C

RN = R * N
RNP = ((RN + N) + NS * L - 1) // (NS * L) * (NS * L)
CNT_PT = RNP // NS

RPT8 = (N // NS) // 8 * 8
TAIL = N - RPT8 * NS

_mesh = plsc.VectorSubcoreMesh(core_axis_name="c", subcore_axis_name="s")


@functools.partial(
    pl.kernel,
    out_type=(
        jax.ShapeDtypeStruct((E_PAD,), jnp.int32),
        jax.ShapeDtypeStruct((E_PAD,), jnp.int32),
        jax.ShapeDtypeStruct((NC, RNP), jnp.float32),
    ),
    mesh=_mesh,
    scratch_types=[
        pltpu.VMEM((C,), jnp.int32),
        pltpu.VMEM((C,), jnp.int32),
        pltpu.VMEM((C,), jnp.int32),
        pltpu.VMEM((C,), jnp.int32),
        pltpu.VMEM((C,), jnp.int32),
        pltpu.VMEM((C,), jnp.float32),
        pltpu.VMEM((CNT_PT,), jnp.float32),
        pltpu.VMEM_SHARED((RNP,), jnp.float32),
    ],
)
def _sc_precompute(src_hbm, dst_hbm, et_hbm, gidx_hbm, sidx_hbm, cnt_hbm,
                   sv, dv, tv, gv, siv, onesv, zv, cnt_sh):
    cid = lax.axis_index("c")
    sid = lax.axis_index("s")
    wid = sid * NC + cid
    base = wid * EW

    for j in range(C // L):
        onesv[pl.ds(j * L, L)] = jnp.ones((L,), jnp.float32)

    def _zb(j, _):
        zv[pl.ds(j * L, L)] = jnp.zeros((L,), jnp.float32)
        return ()
    lax.fori_loop(0, CNT_PT // L, _zb, ())
    pltpu.sync_copy(zv, cnt_sh.at[pl.ds(sid * CNT_PT, CNT_PT)])
    plsc.subcore_barrier()

    def chunk_body(ci, _):
        off = base + ci * C
        pltpu.sync_copy(src_hbm.at[pl.ds(off, C)], sv)
        pltpu.sync_copy(dst_hbm.at[pl.ds(off, C)], dv)
        pltpu.sync_copy(et_hbm.at[pl.ds(off, C)], tv)
        for g in range(C // L):
            sl = pl.ds(g * L, L)
            tn = tv[sl] * N
            gv[sl] = tn + sv[sl]
            siv[sl] = tn + dv[sl]
        pltpu.sync_copy(gv, gidx_hbm.at[pl.ds(off, C)])
        pltpu.sync_copy(siv, sidx_hbm.at[pl.ds(off, C)])
        pltpu.sync_copy(onesv, cnt_sh.at[siv], add=True)
        return ()

    lax.fori_loop(0, NCHUNK, chunk_body, ())
    plsc.subcore_barrier()
    pltpu.sync_copy(cnt_sh.at[pl.ds(sid * CNT_PT, CNT_PT)],
                    cnt_hbm.at[cid, pl.ds(sid * CNT_PT, CNT_PT)])


_INVB = 8192


def _tc_inv_body(cnt_ref, out_ref):
    i = pl.program_id(0)
    col = lax.broadcasted_iota(jnp.int32, (1, _INVB), 1) + i * _INVB
    c = cnt_ref[0:1, :] + cnt_ref[1:2, :]
    inv = 1.0 / jnp.maximum(c, 1.0)
    out_ref[...] = jnp.where(col < RN, inv, 0.0)


def _tc_inv(cnt2):
    return pl.pallas_call(
        _tc_inv_body,
        grid=(RNP // _INVB,),
        in_specs=[pl.BlockSpec((NC, _INVB), lambda i: (0, i))],
        out_specs=pl.BlockSpec((1, _INVB), lambda i: (0, i)),
        out_shape=jax.ShapeDtypeStruct((1, RNP), jnp.float32),
    )(cnt2)


_BN = 400
_NB = N // _BN


def _tc_matmul_body(x_ref, w_ref, xall_ref, rh_ref):
    r = pl.program_id(1)
    prod = jnp.dot(x_ref[...], w_ref[0], preferred_element_type=jnp.float32)
    xall_ref[...] = prod

    @pl.when(r == R)
    def _():
        rh_ref[...] = prod * 0.5


def _tc_matmul(x, wcat):
    return pl.pallas_call(
        _tc_matmul_body,
        grid=(_NB, R + 1),
        in_specs=[
            pl.BlockSpec((_BN, D), lambda i, r: (i, 0)),
            pl.BlockSpec((1, D, D), lambda i, r: (r, 0, 0)),
        ],
        out_specs=[
            pl.BlockSpec((_BN, D), lambda i, r: (r * _NB + i, 0)),
            pl.BlockSpec((_BN, D), lambda i, r: (i, 0)),
        ],
        out_shape=[
            jax.ShapeDtypeStruct(((R + 1) * N, D), jnp.float32),
            jax.ShapeDtypeStruct((N, D), jnp.float32),
        ],
    )(x, wcat)


@functools.partial(
    pl.kernel,
    out_type=jax.ShapeDtypeStruct((NC, N, D), jnp.float32),
    mesh=_mesh,
    scratch_types=[
        pltpu.VMEM((C,), jnp.int32),
        pltpu.VMEM((C,), jnp.int32),
        pltpu.VMEM((C,), jnp.int32),
        pltpu.VMEM((C,), jnp.float32),
        pltpu.VMEM((C, D), jnp.float32),
        pltpu.VMEM_SHARED((N, D), jnp.float32),
        pltpu.SemaphoreType.DMA,
        pltpu.SemaphoreType.DMA,
    ],
)
def _sc_edge(xall_hbm, gidx_hbm, dst_hbm, sidx_hbm, invt_hbm, rh_hbm, acc_hbm,
             gv, dv, siv, iv, rows, acc_sh, sem, sem2):
    cid = lax.axis_index("c")
    sid = lax.axis_index("s")
    wid = sid * NC + cid
    base = wid * EW

    pltpu.sync_copy(rh_hbm.at[pl.ds(sid * RPT8, RPT8)],
                    acc_sh.at[pl.ds(sid * RPT8, RPT8)])

    @pl.when(sid == 0)
    def _():
        pltpu.sync_copy(rh_hbm.at[pl.ds(NS * RPT8, TAIL)],
                        acc_sh.at[pl.ds(NS * RPT8, TAIL)])

    plsc.subcore_barrier()

    def chunk_body(ci, _):
        off = base + ci * C
        pltpu.sync_copy(gidx_hbm.at[pl.ds(off, C)], gv)
        pltpu.sync_copy(dst_hbm.at[pl.ds(off, C)], dv)
        pltpu.sync_copy(sidx_hbm.at[pl.ds(off, C)], siv)
        rows_dma = pltpu.async_copy(xall_hbm.at[gv], rows, sem)
        inv_dma = pltpu.async_copy(invt_hbm.at[siv], iv, sem2)
        rows_dma.wait()
        inv_dma.wait()

        def scale_body(g, _):
            sv = iv[pl.ds(g * L, L)]
            for k in range(L):
                e = g * L + k
                s = sv[k]
                for j in range(D // L):
                    sl = pl.ds(j * L, L)
                    rows[e, sl] = rows[e, sl] * s
            return ()

        lax.fori_loop(0, C // L, scale_body, ())
        pltpu.sync_copy(rows, acc_sh.at[dv], add=True)
        return ()

    lax.fori_loop(0, NCHUNK, chunk_body, ())
    plsc.subcore_barrier()
    pltpu.sync_copy(acc_sh.at[pl.ds(sid * RPT8, RPT8)],
                    acc_hbm.at[cid, pl.ds(sid * RPT8, RPT8)])

    @pl.when(sid == 0)
    def _():
        pltpu.sync_copy(acc_sh.at[pl.ds(NS * RPT8, TAIL)],
                        acc_hbm.at[cid, pl.ds(NS * RPT8, TAIL)])


def _tc_combine_body(acc_ref, b_ref, h_ref):
    a = acc_ref[0] + acc_ref[1] + b_ref[0]
    h_ref[...] = jnp.maximum(a, 0.0)


def _tc_combine(accs, b):
    return pl.pallas_call(
        _tc_combine_body,
        grid=(_NB,),
        in_specs=[
            pl.BlockSpec((NC, _BN, D), lambda i: (0, i, 0)),
            pl.BlockSpec((1, D), lambda i: (0, 0)),
        ],
        out_specs=pl.BlockSpec((_BN, D), lambda i: (i, 0)),
        out_shape=jax.ShapeDtypeStruct((N, D), jnp.float32),
    )(accs, b.reshape(1, D))


def _tc_final_body(acc_ref, b_ref, wout_ref, bout_ref, out_ref):
    a = acc_ref[0] + acc_ref[1] + b_ref[0]
    h = jnp.maximum(a, 0.0)
    out_ref[...] = (jnp.dot(h, wout_ref[...], preferred_element_type=jnp.float32)
                    + bout_ref[0, 0])


def _tc_final(accs, b, wout, bout):
    return pl.pallas_call(
        _tc_final_body,
        grid=(_NB,),
        in_specs=[
            pl.BlockSpec((NC, _BN, D), lambda i: (0, i, 0)),
            pl.BlockSpec((1, D), lambda i: (0, 0)),
            pl.BlockSpec((D, 1), lambda i: (0, 0)),
            pl.BlockSpec((1, 1), lambda i: (0, 0)),
        ],
        out_specs=pl.BlockSpec((_BN, 1), lambda i: (i, 0)),
        out_shape=jax.ShapeDtypeStruct((N, 1), jnp.float32),
    )(accs, b.reshape(1, D), wout, bout.reshape(1, 1))


def kernel(x, edge_index, edge_type, W0, root0, b0, W1, root1, b1, Wout, bout):
    pad = E_PAD - E
    src_p = jnp.concatenate([edge_index[0], jnp.zeros((pad,), jnp.int32)])
    dst_p = jnp.concatenate([edge_index[1], jnp.zeros((pad,), jnp.int32)])
    et_p = jnp.concatenate([edge_type, jnp.full((pad,), R, jnp.int32)])

    gidx, sidx, cnt2 = _sc_precompute(src_p, dst_p, et_p)
    invt = _tc_inv(cnt2).reshape(RNP)

    wcat0 = jnp.concatenate([W0, root0[None]], axis=0)
    wcat1 = jnp.concatenate([W1, root1[None]], axis=0)

    xall0, rh0 = _tc_matmul(x, wcat0)
    accs0 = _sc_edge(xall0, gidx, dst_p, sidx, invt, rh0)
    h1 = _tc_combine(accs0, b0)

    xall1, rh1 = _tc_matmul(h1, wcat1)
    accs1 = _sc_edge(xall1, gidx, dst_p, sidx, invt, rh1)
    out2d = _tc_final(accs1, b1, Wout, bout)
    return out2d.reshape(N)
